# baseline (speedup 1.0000x reference)
import dataclasses
import functools

import jax
import jax.numpy as jnp
from jax.experimental import pallas as pl
from jax.experimental.pallas import tpu as pltpu
from jax.experimental.pallas import tpu_sc as plsc

N = 1048576
DIM = 64
BINS = (1, 2, 3, 4, 8, 16, 32, 64)
CH = 128
LANES = 16
M = N // 2


def _bucket(v):
    acc = (v >= BINS[0]).astype(jnp.int32)
    for b in BINS[1:]:
        acc += (v >= b).astype(jnp.int32)
    return acc


def kernel(lengths, table):
    lengths2 = lengths.astype(jnp.int32).reshape(M, 2).T
    tpair = jnp.concatenate(
        [jnp.repeat(table, 9, axis=0), jnp.tile(table, (9, 1))], axis=1
    )

    mesh = plsc.VectorSubcoreMesh(core_axis_name="c", subcore_axis_name="s")
    cp = pltpu.CompilerParams()
    if "needs_layout_passes" in pltpu.CompilerParams.__dataclass_fields__:
        cp = dataclasses.replace(cp, needs_layout_passes=False)
    cp = dataclasses.replace(cp, use_tc_tiling_on_sc=False)

    @functools.partial(
        pl.kernel,
        out_type=jax.ShapeDtypeStruct((M, 2 * DIM), jnp.float32),
        mesh=mesh,
        scratch_types=[
            pltpu.VMEM((1, CH), jnp.int32),
            pltpu.VMEM_SHARED((81, 2 * DIM), jnp.float32),
        ],
        compiler_params=cp,
    )
    def k(len_hbm, tab_hbm, out_hbm, idx_v, tab_v):
        pltpu.sync_copy(tab_hbm, tab_v)

        def body(len_vmem, out_vmem):
            @pl.loop(0, CH, step=LANES)
            def _(c):
                a = _bucket(len_vmem[0, pl.ds(c, LANES)])
                b = _bucket(len_vmem[1, pl.ds(c, LANES)])
                idx_v[0, pl.ds(c, LANES)] = a * 9 + b

            pltpu.sync_copy(tab_v.at[idx_v.at[0]], out_vmem)

        pltpu.emit_pipeline(
            body,
            grid=(M // CH,),
            in_specs=[pl.BlockSpec((2, CH), lambda i: (0, i))],
            out_specs=[pl.BlockSpec((CH, 2 * DIM), lambda i: (i, 0))],
            core_axis_name=("c", "s"),
            dimension_semantics=(pltpu.PARALLEL,),
        )(len_hbm, out_hbm)

    return k(lengths2, tpair).reshape(N, DIM)


# speedup vs baseline: 1.5155x; 1.5155x over previous
import dataclasses
import functools

import jax
import jax.numpy as jnp
from jax import lax
from jax.experimental import pallas as pl
from jax.experimental.pallas import tpu as pltpu
from jax.experimental.pallas import tpu_sc as plsc

N = 1048576
DIM = 64
BINS = (1, 2, 3, 4, 8, 16, 32, 64)
CH = 128
LANES = 16
M = N // 2


def _bucket(v):
    acc = (v >= BINS[0]).astype(jnp.int32)
    for b in BINS[1:]:
        acc += (v >= b).astype(jnp.int32)
    return acc


def _swap_adjacent(x):
    perm = lax.iota(jnp.int32, LANES) ^ 1
    dnums = lax.GatherDimensionNumbers(
        offset_dims=(), collapsed_slice_dims=(0,), start_index_map=(0,)
    )
    return lax.gather(
        x,
        perm.reshape(LANES, 1),
        dnums,
        slice_sizes=(1,),
        mode=lax.GatherScatterMode.PROMISE_IN_BOUNDS,
    )


def kernel(lengths, table):
    lengths = lengths.astype(jnp.int32).reshape(1, N)
    tpair = jnp.concatenate(
        [jnp.repeat(table, 9, axis=0), jnp.tile(table, (9, 1))], axis=1
    )

    mesh = plsc.VectorSubcoreMesh(core_axis_name="c", subcore_axis_name="s")
    cp = pltpu.CompilerParams()
    if "needs_layout_passes" in pltpu.CompilerParams.__dataclass_fields__:
        cp = dataclasses.replace(cp, needs_layout_passes=False)
    cp = dataclasses.replace(cp, use_tc_tiling_on_sc=False)

    @functools.partial(
        pl.kernel,
        out_type=jax.ShapeDtypeStruct((M, 2 * DIM), jnp.float32),
        mesh=mesh,
        scratch_types=[
            pltpu.VMEM((1, CH + LANES), jnp.int32),
            pltpu.VMEM_SHARED((81, 2 * DIM), jnp.float32),
        ],
        compiler_params=cp,
    )
    def k(len_hbm, tab_hbm, out_hbm, idx_v, tab_v):
        pltpu.sync_copy(tab_hbm, tab_v)

        def body(len_vmem, out_vmem):
            @pl.loop(0, 2 * CH, step=LANES)
            def _(c):
                even = (lax.iota(jnp.int32, LANES) & 1) == 0
                acc = _bucket(len_vmem[0, pl.ds(c, LANES)])
                w = jnp.where(even, acc * 9, acc)
                pair = w + _swap_adjacent(w)
                plsc.store_compressed(
                    idx_v.at[0, pl.ds(c >> 1, LANES)], pair, mask=even
                )

            pltpu.sync_copy(tab_v.at[idx_v.at[0, pl.ds(0, CH)]], out_vmem)

        pltpu.emit_pipeline(
            body,
            grid=(M // CH,),
            in_specs=[pl.BlockSpec((1, 2 * CH), lambda i: (0, i))],
            out_specs=[pl.BlockSpec((CH, 2 * DIM), lambda i: (i, 0))],
            core_axis_name=("c", "s"),
            dimension_semantics=(pltpu.PARALLEL,),
        )(len_hbm, out_hbm)

    return k(lengths, tpair).reshape(N, DIM)


# speedup vs baseline: 1.5167x; 1.0008x over previous
import dataclasses
import functools

import jax
import jax.numpy as jnp
from jax import lax
from jax.experimental import pallas as pl
from jax.experimental.pallas import tpu as pltpu
from jax.experimental.pallas import tpu_sc as plsc

N = 1048576
DIM = 64
BINS = (1, 2, 3, 4, 8, 16, 32, 64)
CH = 128
LANES = 16
M = N // 2


def _bucket(v):
    acc = (v >= BINS[0]).astype(jnp.int32)
    for b in BINS[1:]:
        acc += (v >= b).astype(jnp.int32)
    return acc


def _swap_adjacent(x):
    perm = lax.iota(jnp.int32, LANES) ^ 1
    dnums = lax.GatherDimensionNumbers(
        offset_dims=(), collapsed_slice_dims=(0,), start_index_map=(0,)
    )
    return lax.gather(
        x,
        perm.reshape(LANES, 1),
        dnums,
        slice_sizes=(1,),
        mode=lax.GatherScatterMode.PROMISE_IN_BOUNDS,
    )


def kernel(lengths, table):
    lengths = lengths.astype(jnp.int32).reshape(1, N)
    tpair = jnp.stack(
        [jnp.repeat(table, 9, axis=0), jnp.tile(table, (9, 1))], axis=1
    )

    mesh = plsc.VectorSubcoreMesh(core_axis_name="c", subcore_axis_name="s")
    cp = pltpu.CompilerParams()
    if "needs_layout_passes" in pltpu.CompilerParams.__dataclass_fields__:
        cp = dataclasses.replace(cp, needs_layout_passes=False)
    cp = dataclasses.replace(cp, use_tc_tiling_on_sc=False)

    @functools.partial(
        pl.kernel,
        out_type=jax.ShapeDtypeStruct((M, 2, DIM), jnp.float32),
        mesh=mesh,
        scratch_types=[
            pltpu.VMEM((1, CH + LANES), jnp.int32),
            pltpu.VMEM_SHARED((81, 2, DIM), jnp.float32),
        ],
        compiler_params=cp,
    )
    def k(len_hbm, tab_hbm, out_hbm, idx_v, tab_v):
        pltpu.sync_copy(tab_hbm, tab_v)

        def body(len_vmem, out_vmem):
            @pl.loop(0, 2 * CH, step=LANES)
            def _(c):
                even = (lax.iota(jnp.int32, LANES) & 1) == 0
                acc = _bucket(len_vmem[0, pl.ds(c, LANES)])
                w = jnp.where(even, acc * 9, acc)
                pair = w + _swap_adjacent(w)
                plsc.store_compressed(
                    idx_v.at[0, pl.ds(c >> 1, LANES)], pair, mask=even
                )

            pltpu.sync_copy(
                tab_v.at[idx_v.at[0, pl.ds(0, CH)]], out_vmem
            )

        pltpu.emit_pipeline(
            body,
            grid=(M // CH,),
            in_specs=[pl.BlockSpec((1, 2 * CH), lambda i: (0, i))],
            out_specs=[pl.BlockSpec((CH, 2, DIM), lambda i: (i, 0, 0))],
            core_axis_name=("c", "s"),
            dimension_semantics=(pltpu.PARALLEL,),
        )(len_hbm, out_hbm)

    return k(lengths, tpair).reshape(N, DIM)


# speedup vs baseline: 2.3421x; 1.5442x over previous
import dataclasses
import functools

import jax
import jax.numpy as jnp
from jax.experimental import pallas as pl
from jax.experimental.pallas import tpu as pltpu
from jax.experimental.pallas import tpu_sc as plsc

N = 1048576
DIM = 64
BINS = (1, 2, 3, 4, 8, 16, 32, 64)
LANES = 16
TI = N // 128


def _bucket(v):
    acc = (v >= BINS[0]).astype(jnp.int32)
    for b in BINS[1:]:
        acc += (v >= b).astype(jnp.int32)
    return acc


def kernel(lengths, table):
    lengths = lengths.astype(jnp.int32).reshape(1, N)
    tab_t = table.T.reshape(DIM, 9)

    mesh = plsc.VectorSubcoreMesh(core_axis_name="c", subcore_axis_name="s")
    cp = pltpu.CompilerParams()
    if "needs_layout_passes" in pltpu.CompilerParams.__dataclass_fields__:
        cp = dataclasses.replace(cp, needs_layout_passes=False)
    cp = dataclasses.replace(cp, use_tc_tiling_on_sc=False)

    @functools.partial(
        pl.kernel,
        out_type=jax.ShapeDtypeStruct((8, TI, 8, 128), jnp.float32),
        mesh=mesh,
        scratch_types=[pltpu.VMEM((DIM, 9), jnp.float32)],
        compiler_params=cp,
    )
    def k(len_hbm, tab_hbm, out_hbm, tab_v):
        pltpu.sync_copy(tab_hbm, tab_v)

        def body(len_vmem, out_vmem):
            @pl.loop(0, 128, step=LANES)
            def _(c):
                r = _bucket(len_vmem[0, pl.ds(c, LANES)])
                for td in range(8):
                    for di in range(8):
                        out_vmem[td, 0, di, pl.ds(c, LANES)] = (
                            plsc.load_gather(tab_v.at[td * 8 + di], [r])
                        )

        pltpu.emit_pipeline(
            body,
            grid=(TI,),
            in_specs=[pl.BlockSpec((1, 128), lambda i: (0, i))],
            out_specs=[pl.BlockSpec((8, 1, 8, 128), lambda i: (0, i, 0, 0))],
            core_axis_name=("c", "s"),
            dimension_semantics=(pltpu.PARALLEL,),
        )(len_hbm, out_hbm)

    out4 = k(lengths, tab_t)
    return out4.transpose(1, 3, 0, 2).reshape(N, DIM)


# speedup vs baseline: 8.7750x; 3.7466x over previous
import dataclasses
import functools

import jax
import jax.numpy as jnp
from jax.experimental import pallas as pl
from jax.experimental.pallas import tpu as pltpu
from jax.experimental.pallas import tpu_sc as plsc

N = 1048576
DIM = 64
BINS = (1, 2, 3, 4, 8, 16, 32, 64)
LANES = 16
TI = N // 128


def _bucket(v):
    acc = (v >= BINS[0]).astype(jnp.int32)
    for b in BINS[1:]:
        acc += (v >= b).astype(jnp.int32)
    return acc


def kernel(lengths, table):
    lengths = lengths.astype(jnp.int32).reshape(1, N)
    tab_t = table.T.reshape(DIM, 9)

    mesh = plsc.VectorSubcoreMesh(core_axis_name="c", subcore_axis_name="s")
    cp = pltpu.CompilerParams()
    if "needs_layout_passes" in pltpu.CompilerParams.__dataclass_fields__:
        cp = dataclasses.replace(cp, needs_layout_passes=False)
    cp = dataclasses.replace(cp, use_tc_tiling_on_sc=False)

    @functools.partial(
        pl.kernel,
        out_type=jax.ShapeDtypeStruct((8, TI, 8, 128), jnp.float32),
        mesh=mesh,
        scratch_types=[pltpu.VMEM((DIM, 9), jnp.float32)],
        compiler_params=cp,
    )
    def k(len_hbm, tab_hbm, out_hbm, tab_v):
        pltpu.sync_copy(tab_hbm, tab_v)

        def body(len_vmem, out_vmem):
            @plsc.parallel_loop(0, 128, LANES, unroll=2)
            def _(c):
                r = _bucket(len_vmem[0, pl.ds(c, LANES)])
                for td in range(8):
                    for di in range(8):
                        out_vmem[td, 0, di, pl.ds(c, LANES)] = (
                            plsc.load_gather(tab_v.at[td * 8 + di], [r])
                        )

        pltpu.emit_pipeline(
            body,
            grid=(TI,),
            in_specs=[pl.BlockSpec((1, 128), lambda i: (0, i))],
            out_specs=[pl.BlockSpec((8, 1, 8, 128), lambda i: (0, i, 0, 0))],
            core_axis_name=("c", "s"),
            dimension_semantics=(pltpu.PARALLEL,),
        )(len_hbm, out_hbm)

    out4 = k(lengths, tab_t)
    return out4.transpose(1, 3, 0, 2).reshape(N, DIM)
